# Initial kernel scaffold; baseline (speedup 1.0000x reference)
import jax
import jax.numpy as jnp
from jax.experimental import pallas as pl


def kernel(feature, adj, edge_weight, W1, b1, W2, b2):
    raise NotImplementedError("write your pallas kernel here")


# speedup vs baseline: 7.3163x; 7.3163x over previous
import dataclasses
import functools

import jax
import jax.numpy as jnp
from jax import lax
from jax.experimental import pallas as pl
from jax.experimental.pallas import tpu as pltpu
from jax.experimental.pallas import tpu_sc as plsc

NC = 2
NS = 16
NW = NC * NS
CHUNK = 128


def _sc_compiler_params():
  cp = pltpu.CompilerParams()
  fields = pltpu.CompilerParams.__dataclass_fields__
  if "needs_layout_passes" in fields:
    cp = dataclasses.replace(cp, needs_layout_passes=False)
  if "use_tc_tiling_on_sc" in fields:
    cp = dataclasses.replace(cp, use_tc_tiling_on_sc=False)
  return cp


def _sc_deg(RP, NP):
  rows_per_tile = RP // NW
  nz = NP // NS
  mesh = plsc.VectorSubcoreMesh(core_axis_name="c", subcore_axis_name="s")

  @functools.partial(
      pl.kernel,
      out_type=jax.ShapeDtypeStruct((NC, NP), jnp.float32),
      mesh=mesh,
      scratch_types=[
          pltpu.VMEM((CHUNK,), jnp.int32),
          pltpu.VMEM((CHUNK,), jnp.float32),
          pltpu.VMEM((nz,), jnp.float32),
          pltpu.VMEM_SHARED((NP,), jnp.float32),
      ],
  )
  def k(dst_hbm, ew_hbm, out_hbm, dst_v, ew_v, zb_v, acc_sh):
    c = lax.axis_index("c")
    s = lax.axis_index("s")
    wid = c * NS + s
    row0 = wid * rows_per_tile

    @pl.loop(0, nz // 16)
    def _(i):
      zb_v[pl.ds(i * 16, 16)] = jnp.zeros((16,), jnp.float32)

    pltpu.sync_copy(zb_v, acc_sh.at[pl.ds(s * nz, nz)])
    plsc.subcore_barrier()

    @pl.loop(0, rows_per_tile)
    def _(t):
      r = row0 + t
      pltpu.sync_copy(dst_hbm.at[r], dst_v)
      pltpu.sync_copy(ew_hbm.at[r], ew_v)
      pltpu.sync_copy(ew_v, acc_sh.at[dst_v], add=True)

    plsc.subcore_barrier()
    pltpu.sync_copy(acc_sh.at[pl.ds(s * nz, nz)],
                    out_hbm.at[c, pl.ds(s * nz, nz)])

  return k


def _sc_msg_cols(RP, NP, HALF):
  rows_per_tile = RP // NS
  nz = NP // NS
  zr = 128
  mesh = plsc.VectorSubcoreMesh(core_axis_name="c", subcore_axis_name="s")

  @functools.partial(
      pl.kernel,
      out_type=jax.ShapeDtypeStruct((NC, NP, HALF), jnp.float32),
      mesh=mesh,
      scratch_types=[
          pltpu.VMEM((CHUNK,), jnp.int32),
          pltpu.VMEM((CHUNK,), jnp.int32),
          pltpu.VMEM((CHUNK,), jnp.float32),
          pltpu.VMEM((CHUNK, HALF), jnp.float32),
          pltpu.VMEM((CHUNK, HALF), jnp.float32),
          pltpu.VMEM((zr, HALF), jnp.float32),
          pltpu.VMEM_SHARED((NP, HALF), jnp.float32),
          pltpu.SemaphoreType.DMA,
      ],
      compiler_params=_sc_compiler_params(),
  )
  def k(g_hbm, src_hbm, dst_hbm, ew_hbm, out_hbm,
        src_v, dst_v, ew_v, rows_v, msg_v, zb_v, acc_sh, sem):
    c = lax.axis_index("c")
    s = lax.axis_index("s")
    row0 = s * rows_per_tile
    base = c * NP

    @pl.loop(0, zr)
    def _(i):
      for j in range(HALF // 16):
        zb_v[i, pl.ds(j * 16, 16)] = jnp.zeros((16,), jnp.float32)

    @pl.loop(0, nz // zr)
    def _(t):
      pltpu.sync_copy(zb_v, acc_sh.at[pl.ds(s * nz + t * zr, zr)])
    plsc.subcore_barrier()

    @pl.loop(0, rows_per_tile)
    def _(t):
      r = row0 + t
      pltpu.sync_copy(src_hbm.at[r], src_v)
      pltpu.sync_copy(dst_hbm.at[r], dst_v)
      pltpu.sync_copy(ew_hbm.at[r], ew_v)

      @pl.loop(0, CHUNK // 16)
      def _(g):
        sl = pl.ds(g * 16, 16)
        src_v[sl] = src_v[sl] + base

      pltpu.async_copy(g_hbm.at[src_v], rows_v, sem).wait()

      @pl.loop(0, CHUNK // 16)
      def _(g):
        e0 = g * 16
        for e in range(16):
          scale = plsc.load_gather(ew_v, [jnp.full((16,), e0 + e, jnp.int32)])
          for j in range(HALF // 16):
            sl = pl.ds(j * 16, 16)
            msg_v[e0 + e, sl] = rows_v[e0 + e, sl] * scale

      pltpu.sync_copy(msg_v, acc_sh.at[dst_v], add=True)

    plsc.subcore_barrier()
    pltpu.sync_copy(acc_sh.at[pl.ds(s * nz, nz)],
                    out_hbm.at[c, pl.ds(s * nz, nz)])

  return k


def _sc_msg(RP, NP, W):
  rows_per_tile = RP // NW
  nz = NP // NS
  zr = min(nz, 2048 * 16 // W)
  mesh = plsc.VectorSubcoreMesh(core_axis_name="c", subcore_axis_name="s")

  @functools.partial(
      pl.kernel,
      out_type=jax.ShapeDtypeStruct((NC, NP, W), jnp.float32),
      mesh=mesh,
      scratch_types=[
          pltpu.VMEM((CHUNK,), jnp.int32),
          pltpu.VMEM((CHUNK,), jnp.int32),
          pltpu.VMEM((CHUNK,), jnp.float32),
          pltpu.VMEM((CHUNK, W), jnp.float32),
          pltpu.VMEM((CHUNK, W), jnp.float32),
          pltpu.VMEM((zr, W), jnp.float32),
          pltpu.VMEM_SHARED((NP, W), jnp.float32),
          pltpu.SemaphoreType.DMA,
      ],
      compiler_params=_sc_compiler_params(),
  )
  def k(g_hbm, src_hbm, dst_hbm, ew_hbm, out_hbm,
        src_v, dst_v, ew_v, rows_v, msg_v, zb_v, acc_sh, sem):
    c = lax.axis_index("c")
    s = lax.axis_index("s")
    wid = c * NS + s
    row0 = wid * rows_per_tile

    @pl.loop(0, zr)
    def _(i):
      for j in range(W // 16):
        zb_v[i, pl.ds(j * 16, 16)] = jnp.zeros((16,), jnp.float32)

    @pl.loop(0, nz // zr)
    def _(t):
      pltpu.sync_copy(zb_v, acc_sh.at[pl.ds(s * nz + t * zr, zr)])
    plsc.subcore_barrier()

    @pl.loop(0, rows_per_tile)
    def _(t):
      r = row0 + t
      pltpu.sync_copy(src_hbm.at[r], src_v)
      pltpu.sync_copy(dst_hbm.at[r], dst_v)
      pltpu.sync_copy(ew_hbm.at[r], ew_v)
      pltpu.async_copy(g_hbm.at[src_v], rows_v, sem).wait()

      @pl.loop(0, CHUNK // 16)
      def _(g):
        e0 = g * 16
        for e in range(16):
          scale = plsc.load_gather(ew_v, [jnp.full((16,), e0 + e, jnp.int32)])
          for j in range(W // 16):
            sl = pl.ds(j * 16, 16)
            msg_v[e0 + e, sl] = rows_v[e0 + e, sl] * scale

      pltpu.sync_copy(msg_v, acc_sh.at[dst_v], add=True)

    plsc.subcore_barrier()
    pltpu.sync_copy(acc_sh.at[pl.ds(s * nz, nz)],
                    out_hbm.at[c, pl.ds(s * nz, nz)])

  return k


def _tc1(NP, D, H, RB=1024):
  HALF = H // NC

  def body(feat_ref, w_ref, degp_ref, g1h_ref, dinv_ref):
    deg = degp_ref[0] + degp_ref[1] + 1.0
    dv = lax.rsqrt(deg)
    h = jnp.dot(feat_ref[...], w_ref[...], preferred_element_type=jnp.float32)
    g = h * dv
    for c in range(NC):
      g1h_ref[c] = g[:, c * HALF:(c + 1) * HALF]
    dinv_ref[...] = dv

  return pl.pallas_call(
      body,
      grid=(NP // RB,),
      in_specs=[
          pl.BlockSpec((RB, D), lambda i: (i, 0)),
          pl.BlockSpec((D, H), lambda i: (0, 0)),
          pl.BlockSpec((NC, RB, 1), lambda i: (0, i, 0)),
      ],
      out_specs=[
          pl.BlockSpec((NC, RB, HALF), lambda i: (0, i, 0)),
          pl.BlockSpec((RB, 1), lambda i: (i, 0)),
      ],
      out_shape=[
          jax.ShapeDtypeStruct((NC, NP, HALF), jnp.float32),
          jax.ShapeDtypeStruct((NP, 1), jnp.float32),
      ],
  )


def _tc2(NP, H, C, RB=1024):
  HALF = H // NC

  def body(scat_ref, g1h_ref, dinv_ref, b1_ref, w2_ref, x1_ref, g2_ref):
    dv = dinv_ref[...]
    pre = []
    for c in range(NC):
      b1c = b1_ref[...][:, c * HALF:(c + 1) * HALF]
      pre.append((scat_ref[c] + g1h_ref[c]) * dv + b1c)
    x1 = jnp.maximum(jnp.concatenate(pre, axis=1), 0.0)
    x1_ref[...] = x1
    h2 = jnp.dot(x1, w2_ref[...], preferred_element_type=jnp.float32)
    g2_ref[...] = h2 * dv

  return pl.pallas_call(
      body,
      grid=(NP // RB,),
      in_specs=[
          pl.BlockSpec((NC, RB, HALF), lambda i: (0, i, 0)),
          pl.BlockSpec((NC, RB, HALF), lambda i: (0, i, 0)),
          pl.BlockSpec((RB, 1), lambda i: (i, 0)),
          pl.BlockSpec((1, H), lambda i: (0, 0)),
          pl.BlockSpec((H, C), lambda i: (0, 0)),
      ],
      out_specs=[
          pl.BlockSpec((RB, H), lambda i: (i, 0)),
          pl.BlockSpec((RB, C), lambda i: (i, 0)),
      ],
      out_shape=[
          jax.ShapeDtypeStruct((NP, H), jnp.float32),
          jax.ShapeDtypeStruct((NP, C), jnp.float32),
      ],
  )


def _tc3(NP, C, RB=1024):
  def body(scat_ref, g2_ref, dinv_ref, b2_ref, x2_ref):
    dv = dinv_ref[...]
    x2_ref[...] = (scat_ref[0] + scat_ref[1] + g2_ref[...]) * dv + b2_ref[...]

  return pl.pallas_call(
      body,
      grid=(NP // RB,),
      in_specs=[
          pl.BlockSpec((NC, RB, C), lambda i: (0, i, 0)),
          pl.BlockSpec((RB, C), lambda i: (i, 0)),
          pl.BlockSpec((RB, 1), lambda i: (i, 0)),
          pl.BlockSpec((1, C), lambda i: (0, 0)),
      ],
      out_specs=pl.BlockSpec((RB, C), lambda i: (i, 0)),
      out_shape=jax.ShapeDtypeStruct((NP, C), jnp.float32),
  )


def kernel(feature, adj, edge_weight, W1, b1, W2, b2):
  N, D = feature.shape
  H = W1.shape[1]
  C = W2.shape[1]
  E = edge_weight.shape[0]

  NP = ((N + 2047) // 2048) * 2048
  R = -(-E // CHUNK)
  RP = ((R + NW - 1) // NW) * NW
  EP = RP * CHUNK
  pad_e = EP - E

  src = adj[0].astype(jnp.int32)
  dst = adj[1].astype(jnp.int32)
  pad_dst = N + (jnp.arange(pad_e, dtype=jnp.int32) % (NP - N))
  src_p = jnp.concatenate([src, jnp.zeros((pad_e,), jnp.int32)]).reshape(RP, CHUNK)
  dst_p = jnp.concatenate([dst, pad_dst]).reshape(RP, CHUNK)
  ew_p = jnp.concatenate([edge_weight, jnp.zeros((pad_e,), jnp.float32)]
                         ).reshape(RP, CHUNK)
  feat_p = jnp.pad(feature, ((0, NP - N), (0, 0)))

  HALF = H // NC
  degp = _sc_deg(RP, NP)(dst_p, ew_p)
  g1h, dinv = _tc1(NP, D, H)(feat_p, W1, degp.reshape(NC, NP, 1))
  scat1 = _sc_msg_cols(RP, NP, HALF)(
      g1h.reshape(NC * NP, HALF), src_p, dst_p, ew_p)
  x1_p, g2 = _tc2(NP, H, C)(scat1, g1h, dinv, b1.reshape(1, H), W2)
  scat2 = _sc_msg(RP, NP, C)(g2, src_p, dst_p, ew_p)
  x2_p = _tc3(NP, C)(scat2, g2, dinv, b2.reshape(1, C))

  return (x1_p[:N], x2_p[:N])
